# baseline (speedup 1.0000x reference)
import jax
import jax.numpy as jnp
from jax.experimental import pallas as pl

_H = 512
_W = 512


def _proj_body(xt_ref, k_ref, ind_ref, w_ref):
    x0 = xt_ref[0:1, :]
    x1 = xt_ref[1:2, :]
    x2 = xt_ref[2:3, :]
    px = ((x0 / x2).astype(jnp.bfloat16)).astype(jnp.float32)
    py = ((x1 / x2).astype(jnp.bfloat16)).astype(jnp.float32)
    kb = k_ref[...].astype(jnp.bfloat16).astype(jnp.float32)
    u = jnp.round(kb[0, 0] * px + kb[0, 1] * py + kb[0, 2]).astype(jnp.int32)
    v = jnp.round(kb[1, 0] * px + kb[1, 1] * py + kb[1, 2]).astype(jnp.int32)
    cond = (u > 0) & (u < _W) & (v > 0) & (v < _H) & (x2 > 0.0)
    ind = u + v * _H
    ind_ref[...] = jnp.where(cond, ind, 0)
    w_ref[...] = cond.astype(jnp.float32)


def _project(xf, K):
    B, N, _ = xf.shape
    xt = xf.reshape(B * N, 3).T
    nblk = 16
    blk = (B * N) // nblk
    ind, w = pl.pallas_call(
        _proj_body,
        grid=(nblk,),
        in_specs=[
            pl.BlockSpec((3, blk), lambda i: (0, i)),
            pl.BlockSpec((3, 3), lambda i: (0, 0)),
        ],
        out_specs=(
            pl.BlockSpec((1, blk), lambda i: (0, i)),
            pl.BlockSpec((1, blk), lambda i: (0, i)),
        ),
        out_shape=(
            jax.ShapeDtypeStruct((1, B * N), jnp.int32),
            jax.ShapeDtypeStruct((1, B * N), jnp.float32),
        ),
    )(xt, K)
    return ind.reshape(B, N), w.reshape(B, N)


def kernel(x, c, K):
    orig_batch = x.shape[:-2]
    xf = x.reshape(-1, x.shape[-2], x.shape[-1])
    cf = c.reshape(-1, c.shape[-2], c.shape[-1])
    B, N, _ = xf.shape
    kch = cf.shape[-1]

    ind, w = _project(xf, K)

    flat_ind = (ind + jnp.arange(B, dtype=jnp.int32)[:, None] * (_H * _W)).reshape(-1)
    vals = (cf * w[..., None]).reshape(-1, kch)
    img = jnp.zeros((B * _H * _W, kch), dtype=cf.dtype).at[flat_ind].add(vals)
    acc = jnp.zeros((B * _H * _W,), dtype=cf.dtype).at[flat_ind].add(w.reshape(-1))
    acc = jnp.where(acc == 0, jnp.ones_like(acc), acc)
    img = img / acc[:, None]
    img = img.reshape(B, _H, _W, kch).transpose(0, 3, 1, 2)
    return img.reshape(*orig_batch, kch, _H, _W)
